# baseline (speedup 1.0000x reference)
import functools
import jax
import jax.numpy as jnp
from jax import lax
from jax.experimental import pallas as pl
from jax.experimental.pallas import tpu as pltpu
from jax.experimental.pallas import tpu_sc as plsc

N = 10000
NPAD = 10240
IN_C = 128
HID = 128
OUT_C = 64
BN_EPS = 1e-5

NC, NS = 2, 16
NW = NC * NS
CHUNK = 128
CPT = 80
HCPT = 40
CPT_A = 160
CPT_B = 0
EPAD = NW * CPT * CHUNK
ROWS_PT = NPAD // NS


def _sc_mesh():
    return plsc.VectorSubcoreMesh(core_axis_name="c", subcore_axis_name="s")


def _deg_body(col_hbm, out_hbm, col_v, ones_v, zbuf_v, acc):
    c = lax.axis_index("c")
    s = lax.axis_index("s")
    w = c * NS + s
    pltpu.sync_copy(col_hbm.at[pl.ds(w * CPT, CPT)], col_v)
    one16 = jnp.ones((16,), jnp.float32)
    zero16 = jnp.zeros((16,), jnp.float32)
    for i in range(CHUNK // 16):
        ones_v[pl.ds(i * 16, 16)] = one16

    def zb(i, _):
        zbuf_v[pl.ds(i * 16, 16)] = zero16
        return 0
    lax.fori_loop(0, ROWS_PT // 16, zb, 0)
    pltpu.sync_copy(zbuf_v, acc.at[pl.ds(s * ROWS_PT, ROWS_PT)])
    plsc.subcore_barrier()

    def step(j, _):
        pltpu.sync_copy(ones_v, acc.at[col_v.at[j]], add=True)
        return 0
    lax.fori_loop(0, CPT, step, 0)
    plsc.subcore_barrier()
    pltpu.sync_copy(acc.at[pl.ds(s * ROWS_PT, ROWS_PT)],
                    out_hbm.at[pl.ds(c * NPAD + s * ROWS_PT, ROWS_PT)])


def _sc_degree(col2d):
    return pl.kernel(
        _deg_body,
        out_type=jax.ShapeDtypeStruct((NC * NPAD,), jnp.float32),
        mesh=_sc_mesh(),
        scratch_types=[
            pltpu.VMEM((CPT, CHUNK), jnp.int32),
            pltpu.VMEM((CHUNK,), jnp.float32),
            pltpu.VMEM((ROWS_PT,), jnp.float32),
            pltpu.VMEM_SHARED((NPAD,), jnp.float32),
        ],
    )(col2d)


def _scat_body(d, y_hbm, row_hbm, col_hbm, out_hbm,
               row_v, col_v, g0, g1, acc, semg0, semg1, sems0, sems1):
    c = lax.axis_index("c")
    s = lax.axis_index("s")
    w = c * NS + s

    zero16 = jnp.zeros((16,), jnp.float32)

    def zb(i, _):
        r = i // (d // 16)
        k = i % (d // 16)
        g0[r, pl.ds(k * 16, 16)] = zero16
        return 0
    lax.fori_loop(0, CHUNK * d // 16, zb, 0)
    for k in range(ROWS_PT // CHUNK):
        pltpu.sync_copy(g0, acc.at[pl.ds(s * ROWS_PT + k * CHUNK, CHUNK)])
    plsc.subcore_barrier()

    nh = jnp.where(c == 0, CPT_A // HCPT, CPT_B // HCPT)
    cbase = jnp.where(c == 0, s * CPT_A, NS * CPT_A + s * CPT_B)

    def half(h, _):
        hb = pl.multiple_of(cbase + h * HCPT, 8)
        pltpu.sync_copy(row_hbm.at[pl.ds(hb, HCPT)], row_v)
        pltpu.sync_copy(col_hbm.at[pl.ds(hb, HCPT)], col_v)
        pltpu.async_copy(y_hbm.at[row_v.at[0]], g0, semg0)
        pltpu.async_copy(y_hbm.at[row_v.at[1]], g1, semg1)

        def step(k, _):
            pltpu.make_async_copy(y_hbm.at[row_v.at[2 * k]], g0, semg0).wait()
            pltpu.sync_copy(g0, acc.at[col_v.at[2 * k]], add=True)

            @pl.when(k < HCPT // 2 - 1)
            def _():
                pltpu.async_copy(y_hbm.at[row_v.at[2 * k + 2]], g0, semg0)
            pltpu.make_async_copy(y_hbm.at[row_v.at[2 * k + 1]], g1,
                                  semg1).wait()
            pltpu.sync_copy(g1, acc.at[col_v.at[2 * k + 1]], add=True)

            @pl.when(k < HCPT // 2 - 1)
            def _():
                pltpu.async_copy(y_hbm.at[row_v.at[2 * k + 3]], g1, semg1)
            return 0
        lax.fori_loop(0, HCPT // 2, step, 0)
        return 0
    lax.fori_loop(0, nh, half, 0)
    plsc.subcore_barrier()
    pltpu.sync_copy(acc.at[pl.ds(s * ROWS_PT, ROWS_PT)],
                    out_hbm.at[c, pl.ds(s * ROWS_PT, ROWS_PT)])


def _sc_scatter(y, row2d, col2d, d):
    return pl.kernel(
        functools.partial(_scat_body, d),
        out_type=jax.ShapeDtypeStruct((NC, NPAD, d), jnp.float32),
        mesh=_sc_mesh(),
        scratch_types=[
            pltpu.VMEM((HCPT, CHUNK), jnp.int32),
            pltpu.VMEM((HCPT, CHUNK), jnp.int32),
            pltpu.VMEM((CHUNK, d), jnp.float32),
            pltpu.VMEM((CHUNK, d), jnp.float32),
            pltpu.VMEM_SHARED((NPAD, d), jnp.float32),
            pltpu.SemaphoreType.DMA,
            pltpu.SemaphoreType.DMA,
            pltpu.SemaphoreType.DMA,
            pltpu.SemaphoreType.DMA,
        ],
    )(y, row2d, col2d)


RB = 256
GRID = NPAD // RB


def _dinv_body(dref, oref):
    d = dref[0, :] + dref[1, :] + 1.0
    oref[...] = lax.rsqrt(d)


def _tc_dinv(degpair):
    return pl.pallas_call(
        _dinv_body,
        out_shape=jax.ShapeDtypeStruct((NPAD,), jnp.float32),
    )(degpair)


def _mm1_body(xref, wref, dref, oref):
    xw = jnp.dot(xref[...], wref[...], preferred_element_type=jnp.float32)
    oref[...] = xw * dref[...][:, None]


def _tc_mm1(xp, W1, dinv):
    return pl.pallas_call(
        _mm1_body,
        grid=(GRID,),
        in_specs=[
            pl.BlockSpec((RB, IN_C), lambda i: (i, 0)),
            pl.BlockSpec((IN_C, HID), lambda i: (0, 0)),
            pl.BlockSpec((RB,), lambda i: (i,)),
        ],
        out_specs=pl.BlockSpec((RB, HID), lambda i: (i, 0)),
        out_shape=jax.ShapeDtypeStruct((NPAD, HID), jnp.float32),
    )(xp, W1, dinv)


def _fuse_body(a0, a1, yref, dref, bref, mref, vref, gref, betaref, wref,
               oref):
    dv = dref[...][:, None]
    t = (a0[0] + a1[0] + yref[...]) * dv + bref[...][None, :]
    scale = gref[...] * lax.rsqrt(vref[...] + BN_EPS)
    shift = betaref[...] - mref[...] * scale
    h = jnp.maximum(t * scale[None, :] + shift[None, :], 0.0)
    o = jnp.dot(h, wref[...], preferred_element_type=jnp.float32)
    oref[...] = o * dv


def _tc_fuse(accpair, y, dinv, b, m, v, g, beta, W):
    din = y.shape[1]
    dout = W.shape[1]
    return pl.pallas_call(
        _fuse_body,
        grid=(GRID,),
        in_specs=[
            pl.BlockSpec((1, RB, din), lambda i: (0, i, 0)),
            pl.BlockSpec((1, RB, din), lambda i: (1, i, 0)),
            pl.BlockSpec((RB, din), lambda i: (i, 0)),
            pl.BlockSpec((RB,), lambda i: (i,)),
            pl.BlockSpec((din,), lambda i: (0,)),
            pl.BlockSpec((din,), lambda i: (0,)),
            pl.BlockSpec((din,), lambda i: (0,)),
            pl.BlockSpec((din,), lambda i: (0,)),
            pl.BlockSpec((din,), lambda i: (0,)),
            pl.BlockSpec((din, dout), lambda i: (0, 0)),
        ],
        out_specs=pl.BlockSpec((RB, dout), lambda i: (i, 0)),
        out_shape=jax.ShapeDtypeStruct((NPAD, dout), jnp.float32),
    )(accpair, accpair, y, dinv, b, m, v, g, beta, W)


def _final_body(a0, a1, yref, dref, bref, oref):
    dv = dref[...][:, None]
    t = (a0[0] + a1[0] + yref[...]) * dv + bref[...][None, :]
    t = t[:, :OUT_C]
    mx = jnp.max(t, axis=-1, keepdims=True)
    e = jnp.exp(t - mx)
    lse = jnp.log(jnp.sum(e, axis=-1, keepdims=True)) + mx
    oref[...] = t - lse


def _tc_final(accpair, y, dinv, b):
    din = y.shape[1]
    return pl.pallas_call(
        _final_body,
        grid=(GRID,),
        in_specs=[
            pl.BlockSpec((1, RB, din), lambda i: (0, i, 0)),
            pl.BlockSpec((1, RB, din), lambda i: (1, i, 0)),
            pl.BlockSpec((RB, din), lambda i: (i, 0)),
            pl.BlockSpec((RB,), lambda i: (i,)),
            pl.BlockSpec((din,), lambda i: (0,)),
        ],
        out_specs=pl.BlockSpec((RB, OUT_C), lambda i: (i, 0)),
        out_shape=jax.ShapeDtypeStruct((NPAD, OUT_C), jnp.float32),
    )(accpair, accpair, y, dinv, b)


def kernel(x, edge_index, relations, W1, b1, W2, b2, W3, b3,
           bn1_mean, bn1_var, bn1_g, bn1_b, bn2_mean, bn2_var, bn2_g, bn2_b):
    del relations
    E = edge_index.shape[1]
    pad = EPAD - E
    row2d = jnp.concatenate(
        [edge_index[0], jnp.zeros((pad,), jnp.int32)]).reshape(-1, CHUNK)
    trash = N + jnp.arange(pad, dtype=jnp.int32) % (NPAD - N)
    col2d = jnp.concatenate([edge_index[1], trash]).reshape(-1, CHUNK)
    xp = jnp.pad(x, ((0, NPAD - N), (0, 0)))

    degpair = _sc_degree(col2d).reshape(NC, NPAD)
    dinv = _tc_dinv(degpair)

    y1 = _tc_mm1(xp, W1, dinv)
    acc1 = _sc_scatter(y1, row2d, col2d, HID)
    y2 = _tc_fuse(acc1, y1, dinv, b1, bn1_mean, bn1_var, bn1_g, bn1_b, W2)
    acc2 = _sc_scatter(y2, row2d, col2d, HID)
    W3p = jnp.pad(W3, ((0, 0), (0, HID - OUT_C)))
    b3p = jnp.pad(b3, (0, HID - OUT_C))
    y3 = _tc_fuse(acc2, y2, dinv, b2, bn2_mean, bn2_var, bn2_g, bn2_b, W3p)
    acc3 = _sc_scatter(y3, row2d, col2d, HID)
    z = _tc_final(acc3, y3, dinv, b3p)
    return z[:N]


# speedup vs baseline: 1.2153x; 1.2153x over previous
import functools
import jax
import jax.numpy as jnp
from jax import lax
from jax.experimental import pallas as pl
from jax.experimental.pallas import tpu as pltpu
from jax.experimental.pallas import tpu_sc as plsc

N = 10000
NPAD = 10240
IN_C = 128
HID = 128
OUT_C = 64
BN_EPS = 1e-5

NC, NS = 2, 16
NW = NC * NS
CHUNK = 128
CPT = 80
HCPT = 32
CPT_A = 128
CPT_B = 32
EPAD = NW * CPT * CHUNK
ROWS_PT = NPAD // NS


def _sc_mesh():
    return plsc.VectorSubcoreMesh(core_axis_name="c", subcore_axis_name="s")


def _deg_body(col_hbm, out_hbm, col_v, ones_v, zbuf_v, acc):
    c = lax.axis_index("c")
    s = lax.axis_index("s")
    w = c * NS + s
    pltpu.sync_copy(col_hbm.at[pl.ds(w * CPT, CPT)], col_v)
    one16 = jnp.ones((16,), jnp.float32)
    zero16 = jnp.zeros((16,), jnp.float32)
    for i in range(CHUNK // 16):
        ones_v[pl.ds(i * 16, 16)] = one16

    def zb(i, _):
        zbuf_v[pl.ds(i * 16, 16)] = zero16
        return 0
    lax.fori_loop(0, ROWS_PT // 16, zb, 0)
    pltpu.sync_copy(zbuf_v, acc.at[pl.ds(s * ROWS_PT, ROWS_PT)])
    plsc.subcore_barrier()

    def step(j, _):
        pltpu.sync_copy(ones_v, acc.at[col_v.at[j]], add=True)
        return 0
    lax.fori_loop(0, CPT, step, 0)
    plsc.subcore_barrier()
    pltpu.sync_copy(acc.at[pl.ds(s * ROWS_PT, ROWS_PT)],
                    out_hbm.at[pl.ds(c * NPAD + s * ROWS_PT, ROWS_PT)])


def _sc_degree(col2d):
    return pl.kernel(
        _deg_body,
        out_type=jax.ShapeDtypeStruct((NC * NPAD,), jnp.float32),
        mesh=_sc_mesh(),
        scratch_types=[
            pltpu.VMEM((CPT, CHUNK), jnp.int32),
            pltpu.VMEM((CHUNK,), jnp.float32),
            pltpu.VMEM((ROWS_PT,), jnp.float32),
            pltpu.VMEM_SHARED((NPAD,), jnp.float32),
        ],
    )(col2d)


def _scat_body(d, y_hbm, row_hbm, col_hbm, out_hbm,
               row_v, col_v, g0, g1, acc, semg0, semg1, sems0, sems1):
    c = lax.axis_index("c")
    s = lax.axis_index("s")
    w = c * NS + s

    zero16 = jnp.zeros((16,), jnp.float32)

    def zb(i, _):
        r = i // (d // 16)
        k = i % (d // 16)
        g0[r, pl.ds(k * 16, 16)] = zero16
        return 0
    lax.fori_loop(0, CHUNK * d // 16, zb, 0)
    for k in range(ROWS_PT // CHUNK):
        pltpu.sync_copy(g0, acc.at[pl.ds(s * ROWS_PT + k * CHUNK, CHUNK)])
    plsc.subcore_barrier()

    nh = jnp.where(c == 0, CPT_A // HCPT, CPT_B // HCPT)
    cbase = jnp.where(c == 0, s * CPT_A, NS * CPT_A + s * CPT_B)

    def half(h, _):
        hb = pl.multiple_of(cbase + h * HCPT, 8)
        pltpu.sync_copy(row_hbm.at[pl.ds(hb, HCPT)], row_v)
        pltpu.sync_copy(col_hbm.at[pl.ds(hb, HCPT)], col_v)
        pltpu.async_copy(y_hbm.at[row_v.at[0]], g0, semg0)
        pltpu.async_copy(y_hbm.at[row_v.at[1]], g1, semg1)

        def step(k, _):
            pltpu.make_async_copy(y_hbm.at[row_v.at[2 * k]], g0, semg0).wait()
            pltpu.sync_copy(g0, acc.at[col_v.at[2 * k]], add=True)

            @pl.when(k < HCPT // 2 - 1)
            def _():
                pltpu.async_copy(y_hbm.at[row_v.at[2 * k + 2]], g0, semg0)
            pltpu.make_async_copy(y_hbm.at[row_v.at[2 * k + 1]], g1,
                                  semg1).wait()
            pltpu.sync_copy(g1, acc.at[col_v.at[2 * k + 1]], add=True)

            @pl.when(k < HCPT // 2 - 1)
            def _():
                pltpu.async_copy(y_hbm.at[row_v.at[2 * k + 3]], g1, semg1)
            return 0
        lax.fori_loop(0, HCPT // 2, step, 0)
        return 0
    lax.fori_loop(0, nh, half, 0)
    plsc.subcore_barrier()
    pltpu.sync_copy(acc.at[pl.ds(s * ROWS_PT, ROWS_PT)],
                    out_hbm.at[c, pl.ds(s * ROWS_PT, ROWS_PT)])


def _sc_scatter(y, row2d, col2d, d):
    return pl.kernel(
        functools.partial(_scat_body, d),
        out_type=jax.ShapeDtypeStruct((NC, NPAD, d), jnp.float32),
        mesh=_sc_mesh(),
        scratch_types=[
            pltpu.VMEM((HCPT, CHUNK), jnp.int32),
            pltpu.VMEM((HCPT, CHUNK), jnp.int32),
            pltpu.VMEM((CHUNK, d), jnp.float32),
            pltpu.VMEM((CHUNK, d), jnp.float32),
            pltpu.VMEM_SHARED((NPAD, d), jnp.float32),
            pltpu.SemaphoreType.DMA,
            pltpu.SemaphoreType.DMA,
            pltpu.SemaphoreType.DMA,
            pltpu.SemaphoreType.DMA,
        ],
    )(y, row2d, col2d)


RB = 256
GRID = NPAD // RB


def _dinv_body(dref, oref):
    d = dref[0, :] + dref[1, :] + 1.0
    oref[...] = lax.rsqrt(d)


def _tc_dinv(degpair):
    return pl.pallas_call(
        _dinv_body,
        out_shape=jax.ShapeDtypeStruct((NPAD,), jnp.float32),
    )(degpair)


def _mm1_body(xref, wref, dref, oref):
    xw = jnp.dot(xref[...], wref[...], preferred_element_type=jnp.float32)
    oref[...] = xw * dref[...][:, None]


def _tc_mm1(xp, W1, dinv):
    return pl.pallas_call(
        _mm1_body,
        grid=(GRID,),
        in_specs=[
            pl.BlockSpec((RB, IN_C), lambda i: (i, 0)),
            pl.BlockSpec((IN_C, HID), lambda i: (0, 0)),
            pl.BlockSpec((RB,), lambda i: (i,)),
        ],
        out_specs=pl.BlockSpec((RB, HID), lambda i: (i, 0)),
        out_shape=jax.ShapeDtypeStruct((NPAD, HID), jnp.float32),
    )(xp, W1, dinv)


def _fuse_body(a0, a1, yref, dref, bref, mref, vref, gref, betaref, wref,
               oref):
    dv = dref[...][:, None]
    t = (a0[0] + a1[0] + yref[...]) * dv + bref[...][None, :]
    scale = gref[...] * lax.rsqrt(vref[...] + BN_EPS)
    shift = betaref[...] - mref[...] * scale
    h = jnp.maximum(t * scale[None, :] + shift[None, :], 0.0)
    o = jnp.dot(h, wref[...], preferred_element_type=jnp.float32)
    oref[...] = o * dv


def _tc_fuse(accpair, y, dinv, b, m, v, g, beta, W):
    din = y.shape[1]
    dout = W.shape[1]
    return pl.pallas_call(
        _fuse_body,
        grid=(GRID,),
        in_specs=[
            pl.BlockSpec((1, RB, din), lambda i: (0, i, 0)),
            pl.BlockSpec((1, RB, din), lambda i: (1, i, 0)),
            pl.BlockSpec((RB, din), lambda i: (i, 0)),
            pl.BlockSpec((RB,), lambda i: (i,)),
            pl.BlockSpec((din,), lambda i: (0,)),
            pl.BlockSpec((din,), lambda i: (0,)),
            pl.BlockSpec((din,), lambda i: (0,)),
            pl.BlockSpec((din,), lambda i: (0,)),
            pl.BlockSpec((din,), lambda i: (0,)),
            pl.BlockSpec((din, dout), lambda i: (0, 0)),
        ],
        out_specs=pl.BlockSpec((RB, dout), lambda i: (i, 0)),
        out_shape=jax.ShapeDtypeStruct((NPAD, dout), jnp.float32),
    )(accpair, accpair, y, dinv, b, m, v, g, beta, W)


def _final_body(a0, a1, yref, dref, bref, oref):
    dv = dref[...][:, None]
    t = (a0[0] + a1[0] + yref[...]) * dv + bref[...][None, :]
    t = t[:, :OUT_C]
    mx = jnp.max(t, axis=-1, keepdims=True)
    e = jnp.exp(t - mx)
    lse = jnp.log(jnp.sum(e, axis=-1, keepdims=True)) + mx
    oref[...] = t - lse


def _tc_final(accpair, y, dinv, b):
    din = y.shape[1]
    return pl.pallas_call(
        _final_body,
        grid=(GRID,),
        in_specs=[
            pl.BlockSpec((1, RB, din), lambda i: (0, i, 0)),
            pl.BlockSpec((1, RB, din), lambda i: (1, i, 0)),
            pl.BlockSpec((RB, din), lambda i: (i, 0)),
            pl.BlockSpec((RB,), lambda i: (i,)),
            pl.BlockSpec((din,), lambda i: (0,)),
        ],
        out_specs=pl.BlockSpec((RB, OUT_C), lambda i: (i, 0)),
        out_shape=jax.ShapeDtypeStruct((NPAD, OUT_C), jnp.float32),
    )(accpair, accpair, y, dinv, b)


def kernel(x, edge_index, relations, W1, b1, W2, b2, W3, b3,
           bn1_mean, bn1_var, bn1_g, bn1_b, bn2_mean, bn2_var, bn2_g, bn2_b):
    del relations
    E = edge_index.shape[1]
    pad = EPAD - E
    row2d = jnp.concatenate(
        [edge_index[0], jnp.zeros((pad,), jnp.int32)]).reshape(-1, CHUNK)
    trash = N + jnp.arange(pad, dtype=jnp.int32) % (NPAD - N)
    col2d = jnp.concatenate([edge_index[1], trash]).reshape(-1, CHUNK)
    xp = jnp.pad(x, ((0, NPAD - N), (0, 0)))

    degpair = _sc_degree(col2d).reshape(NC, NPAD)
    dinv = _tc_dinv(degpair)

    y1 = _tc_mm1(xp, W1, dinv)
    acc1 = _sc_scatter(y1, row2d, col2d, HID)
    y2 = _tc_fuse(acc1, y1, dinv, b1, bn1_mean, bn1_var, bn1_g, bn1_b, W2)
    acc2 = _sc_scatter(y2, row2d, col2d, HID)
    W3p = jnp.pad(W3, ((0, 0), (0, HID - OUT_C)))
    b3p = jnp.pad(b3, (0, HID - OUT_C))
    y3 = _tc_fuse(acc2, y2, dinv, b2, bn2_mean, bn2_var, bn2_g, bn2_b, W3p)
    acc3 = _sc_scatter(y3, row2d, col2d, HID)
    z = _tc_final(acc3, y3, dinv, b3p)
    return z[:N]
